# baseline (speedup 1.0000x reference)
import functools

import jax
import jax.numpy as jnp
import numpy as np
from jax import lax
from jax.experimental import pallas as pl
from jax.experimental.pallas import tpu as pltpu
from jax.experimental.pallas import tpu_sc as plsc

_MAX_RADIUS = 1.3

_NC = 2
_NS = 16
_NW = _NC * _NS
_CH = 128


def _silu_c():
    z = np.linspace(-12.0, 12.0, 200001)
    phi = np.exp(-0.5 * z * z) / np.sqrt(2.0 * np.pi)
    s = z / (1.0 + np.exp(-z))
    return float(1.0 / np.sqrt(np.trapz(s * s * phi, z)))


_SILU_C = _silu_c()


def _table_body(f_ref, wq_ref, dw_ref, o_ref):
    f = f_ref[...]
    q = jnp.dot(f, wq_ref[...], preferred_element_type=jnp.float32)
    qd = jnp.dot(q, dw_ref[...], preferred_element_type=jnp.float32) * 0.25
    o_ref[...] = jnp.concatenate([f, qd], axis=1)


def _make_table(f, Wq, dot_W, interpret=False):
    n, mul = f.shape
    return pl.pallas_call(
        _table_body,
        out_shape=jax.ShapeDtypeStruct((n, 2 * mul), jnp.float32),
        interpret=interpret,
    )(f, Wq, dot_W)


def _edge_body(nedges, mul, eleT_ref, td_ref, fs_ref, shT_ref, lenT_ref,
               w1kT_ref, w1vT_ref, wc_ref, o_ref):
    blk = eleT_ref.shape[1]
    nb = eleT_ref.shape[0]
    npack = 128 // (2 * mul)
    q = blk // npack
    ele = eleT_ref[...]
    hk = jax.nn.silu(jnp.dot(w1kT_ref[...], ele,
                             preferred_element_type=jnp.float32)
                     * (1.0 / np.sqrt(nb))) * _SILU_C
    hv = jax.nn.silu(jnp.dot(w1vT_ref[...], ele,
                             preferred_element_type=jnp.float32)
                     * (1.0 / np.sqrt(nb))) * _SILU_C
    tdT = td_ref[...].T
    fsT = fs_ref[...].T
    fd = jnp.concatenate(
        [tdT[2 * mul * s:2 * mul * s + mul, :] for s in range(npack)], axis=1)
    qdd = jnp.concatenate(
        [tdT[2 * mul * s + mul:2 * mul * (s + 1), :] for s in range(npack)],
        axis=1)
    fs = jnp.concatenate(
        [fsT[2 * mul * s:2 * mul * s + mul, :] for s in range(npack)], axis=1)
    a_k = jnp.repeat(hk, mul, axis=0) * jnp.tile(fd, (mul, 1))
    a_v = jnp.repeat(hv, mul, axis=0) * jnp.tile(fs, (mul, 1))
    kv = jnp.dot(wc_ref[...], jnp.concatenate([a_k, a_v], axis=0),
                 preferred_element_type=jnp.float32)
    kv = kv * (shT_ref[...] * (1.0 / mul))
    k = kv[:mul, :]
    v = kv[mul:, :]
    logit = jnp.sum(qdd * k, axis=0, keepdims=True) * (1.0 / mul)
    x = 10.0 * (1.0 - lenT_ref[...] / _MAX_RADIUS)
    xs = jnp.where(x > 0.0, x, 1.0)
    cutoff = jnp.where(x > 0.0, jnp.exp(-1.0 / xs), 0.0)
    w = cutoff * jnp.exp(logit)
    jcol = lax.broadcasted_iota(jnp.int32, (1, blk), 1)
    oid = pl.program_id(0) * blk + npack * (jcol % q) + jcol // q
    w = jnp.where(oid < nedges, w, 0.0)
    u = jnp.sqrt(w) * v
    out32 = jnp.concatenate([u, jnp.broadcast_to(w, (mul, blk))], axis=0)
    stacked = jnp.concatenate(
        [out32[:, s * q:(s + 1) * q] for s in range(npack)], axis=0)
    o_ref[...] = stacked.T


def _edge_stage(eleT, td_pk, fs_pk, shT, lenT, W1k, W2k, W1v, W2v, nedges,
                blk=4096, interpret=False):
    nb, epad = eleT.shape
    mul = td_pk.shape[1] // 8
    hid = W1k.shape[1]
    w2kT = jnp.transpose(W2k.reshape(hid, mul, mul), (2, 0, 1)).reshape(mul, hid * mul)
    w2vT = jnp.transpose(W2v.reshape(hid, mul, mul), (2, 0, 1)).reshape(mul, hid * mul)
    zero = jnp.zeros((mul, hid * mul), jnp.float32)
    wc = jnp.concatenate([
        jnp.concatenate([w2kT, zero], axis=1),
        jnp.concatenate([zero, w2vT], axis=1),
    ], axis=0)
    grid = (epad // blk,)
    npack = 128 // (2 * mul)
    q = blk // npack
    return pl.pallas_call(
        functools.partial(_edge_body, nedges, mul),
        grid=grid,
        in_specs=[
            pl.BlockSpec((nb, blk), lambda i: (0, i)),
            pl.BlockSpec((q, 2 * mul * npack), lambda i: (i, 0)),
            pl.BlockSpec((q, 2 * mul * npack), lambda i: (i, 0)),
            pl.BlockSpec((1, blk), lambda i: (0, i)),
            pl.BlockSpec((1, blk), lambda i: (0, i)),
            pl.BlockSpec((mul, nb), lambda i: (0, 0)),
            pl.BlockSpec((mul, nb), lambda i: (0, 0)),
            pl.BlockSpec((2 * mul, 2 * hid * mul), lambda i: (0, 0)),
        ],
        out_specs=pl.BlockSpec((q, 2 * mul * npack), lambda i: (i, 0)),
        out_shape=jax.ShapeDtypeStruct((epad // npack, 2 * mul * npack),
                                       jnp.float32),
        interpret=interpret,
    )(eleT, td_pk, fs_pk, shT, lenT, W1k.T, W1v.T, wc)


def _norm_body(p0_ref, p1_ref, o_ref):
    mul = o_ref.shape[1]
    s = p0_ref[:, :mul] + p1_ref[:, :mul]
    z = p0_ref[:, mul:mul + 1] + p1_ref[:, mul:mul + 1]
    z = jnp.where(z == 0.0, 1.0, z)
    o_ref[...] = s * lax.rsqrt(z)


def _normalize(p0, p1, interpret=False):
    n = p0.shape[0]
    mul = p0.shape[1] // 2
    return pl.pallas_call(
        _norm_body,
        out_shape=jax.ShapeDtypeStruct((n, mul), jnp.float32),
        interpret=interpret,
    )(p0, p1)


def _gather_calls(table, dst3, src3, epad):
    n, tw = table.shape
    nw, j, ch = dst3.shape
    per = j * ch
    mesh = plsc.VectorSubcoreMesh(core_axis_name="c", subcore_axis_name="s")

    cpg = 5 if j % 5 == 0 else (4 if j % 4 == 0 else 1)
    ng = j // cpg
    rows_g = cpg * ch

    @functools.partial(
        pl.kernel, mesh=mesh,
        compiler_params=pltpu.CompilerParams(use_tc_tiling_on_sc=False),
        out_type=(jax.ShapeDtypeStruct((epad, tw), jnp.float32),
                  jax.ShapeDtypeStruct((epad, tw), jnp.float32)),
        scratch_types=[
            pltpu.VMEM((j, ch), jnp.int32),
            pltpu.VMEM((j, ch), jnp.int32),
            pltpu.VMEM((2 * rows_g, tw), jnp.float32),
            pltpu.VMEM((2 * rows_g, tw), jnp.float32),
            pltpu.SemaphoreType.DMA,
            pltpu.SemaphoreType.DMA,
        ],
    )
    def gk(t_hbm, d_hbm, s_hbm, td_out, fs_out,
           idx_d, idx_s, bt, bf, sg, sw):
        wid = lax.axis_index("s") * _NC + lax.axis_index("c")
        pltpu.sync_copy(d_hbm.at[wid], idx_d)
        pltpu.sync_copy(s_hbm.at[wid], idx_s)

        def issue_gets(g, b):
            gets = []
            for c in range(cpg):
                jj = g * cpg + c
                off = b * rows_g + c * ch
                gets.append(pltpu.async_copy(
                    t_hbm.at[idx_d.at[jj]], bt.at[pl.ds(off, ch)], sg))
                gets.append(pltpu.async_copy(
                    t_hbm.at[idx_s.at[jj]], bf.at[pl.ds(off, ch)], sg))
            return gets

        gets = {0: issue_gets(0, 0), 1: []}
        writes = {0: [], 1: []}
        for g in range(ng):
            b = g % 2
            o = (g + 1) % 2
            if g + 1 < ng:
                for h in writes[o]:
                    h.wait()
                gets[o] = issue_gets(g + 1, o)
            for h in gets[b]:
                h.wait()
            base = wid * per + g * rows_g
            writes[b] = [
                pltpu.async_copy(bt.at[pl.ds(b * rows_g, rows_g)],
                                 td_out.at[pl.ds(base, rows_g)], sw),
                pltpu.async_copy(bf.at[pl.ds(b * rows_g, rows_g)],
                                 fs_out.at[pl.ds(base, rows_g)], sw),
            ]
        for b in (0, 1):
            for h in writes[b]:
                h.wait()

    return gk(table, dst3, src3)


def _scatter_call(rows, dst3, zeros, n):
    epad, tw = rows.shape
    nw, j, ch = dst3.shape
    per = j * ch
    mesh = plsc.VectorSubcoreMesh(core_axis_name="c", subcore_axis_name="s")

    cpg = 8 if j % 8 == 0 else (10 if j % 10 == 0 else (5 if j % 5 == 0 else 1))
    ng = j // cpg
    rows_g = cpg * ch

    @functools.partial(
        pl.kernel, mesh=mesh,
        compiler_params=pltpu.CompilerParams(use_tc_tiling_on_sc=False),
        out_type=jax.ShapeDtypeStruct((_NC, n, tw), jnp.float32),
        scratch_types=[
            pltpu.VMEM((j, ch), jnp.int32),
            pltpu.VMEM((2 * rows_g, tw), jnp.float32),
            pltpu.VMEM_SHARED((n, tw), jnp.float32),
            pltpu.SemaphoreType.DMA,
            pltpu.SemaphoreType.DMA,
        ],
    )
    def sk(r_hbm, d_hbm, z_hbm, p_out, idx_v, buf, acc, sr, ss):
        cid = lax.axis_index("c")
        sid = lax.axis_index("s")
        wid = sid * _NC + cid
        pltpu.sync_copy(d_hbm.at[wid], idx_v)

        @pl.when(sid == 0)
        def _():
            pltpu.sync_copy(z_hbm, acc)

        plsc.subcore_barrier()

        def issue_read(g, b):
            base = wid * per + g * rows_g
            return [pltpu.async_copy(
                r_hbm.at[pl.ds(base, rows_g)],
                buf.at[pl.ds(b * rows_g, rows_g)], sr)]

        reads = {0: issue_read(0, 0), 1: []}
        scats = {0: [], 1: []}
        for g in range(ng):
            b = g % 2
            o = (g + 1) % 2
            if g + 1 < ng:
                for h in scats[o]:
                    h.wait()
                reads[o] = issue_read(g + 1, o)
            for h in reads[b]:
                h.wait()
            scats[b] = []
            for c in range(cpg):
                jj = g * cpg + c
                scats[b].append(pltpu.async_copy(
                    buf.at[pl.ds(b * rows_g + c * ch, ch)],
                    acc.at[idx_v.at[jj]], ss, add=True))
        for b in (0, 1):
            for h in scats[b]:
                h.wait()
        plsc.subcore_barrier()

        @pl.when(sid == 0)
        def _():
            pltpu.sync_copy(acc, p_out.at[cid])

    return sk(rows, dst3, zeros)


def kernel(f, edge_index, edge_length, edge_sh, edge_length_embedded,
           Wq, fck_W1, fck_W2, fcv_W1, fcv_W2, dot_W):
    n, mul = f.shape
    e = edge_index.shape[1]
    nb = edge_length_embedded.shape[1]
    grain = _NW * _CH
    epad = ((e + grain - 1) // grain) * grain
    pad = epad - e
    per = epad // _NW
    j = per // _CH

    src = jnp.pad(edge_index[0], (0, pad)).reshape(_NW, j, _CH)
    dst = jnp.pad(edge_index[1], (0, pad)).reshape(_NW, j, _CH)

    blk = 4096
    npack = 128 // (2 * mul)
    q = blk // npack
    jloc = np.arange(blk)
    perm_blk = npack * (jloc % q) + jloc // q
    perm = (np.arange(epad) // blk) * blk + np.tile(perm_blk, epad // blk)
    perm = jnp.asarray(perm, jnp.int32)

    eleT = jnp.pad(edge_length_embedded, ((0, pad), (0, 0)))[perm].T
    lenT = jnp.pad(edge_length, (0, pad))[perm].reshape(1, epad)
    shT = jnp.pad(edge_sh[:, 0], (0, pad))[perm].reshape(1, epad)

    table = _make_table(f, Wq, dot_W)
    td, fs = _gather_calls(table, dst, src, epad)
    rows_pk = _edge_stage(eleT, td.reshape(epad // npack, 128),
                          fs.reshape(epad // npack, 128), shT, lenT,
                          fck_W1, fck_W2, fcv_W1, fcv_W2, e, blk=blk)
    parts = _scatter_call(rows_pk.reshape(epad, 2 * mul), dst,
                          jnp.zeros((n, 2 * mul), jnp.float32), n)
    return _normalize(parts[0], parts[1])


# speedup vs baseline: 1.3020x; 1.3020x over previous
import functools

import jax
import jax.numpy as jnp
import numpy as np
from jax import lax
from jax.experimental import pallas as pl
from jax.experimental.pallas import tpu as pltpu
from jax.experimental.pallas import tpu_sc as plsc

_MAX_RADIUS = 1.3

_NC = 2
_NS = 16
_NW = _NC * _NS
_CH = 128


def _silu_c():
    z = np.linspace(-12.0, 12.0, 200001)
    phi = np.exp(-0.5 * z * z) / np.sqrt(2.0 * np.pi)
    s = z / (1.0 + np.exp(-z))
    return float(1.0 / np.sqrt(np.trapz(s * s * phi, z)))


_SILU_C = _silu_c()


def _table_body(f_ref, wq_ref, dw_ref, o_ref):
    f = f_ref[...]
    q = jnp.dot(f, wq_ref[...], preferred_element_type=jnp.float32)
    qd = jnp.dot(q, dw_ref[...], preferred_element_type=jnp.float32) * 0.25
    o_ref[...] = jnp.concatenate([f, qd], axis=1)


def _make_table(f, Wq, dot_W, interpret=False):
    n, mul = f.shape
    return pl.pallas_call(
        _table_body,
        out_shape=jax.ShapeDtypeStruct((n, 2 * mul), jnp.float32),
        interpret=interpret,
    )(f, Wq, dot_W)


def _edge_body(nedges, mul, eleT_ref, td_ref, fs_ref, shT_ref, lenT_ref,
               w1kT_ref, w1vT_ref, wc_ref, o_ref):
    blk = eleT_ref.shape[1]
    nb = eleT_ref.shape[0]
    npack = 128 // (2 * mul)
    q = blk // npack
    ele = eleT_ref[...]
    hk = jax.nn.silu(jnp.dot(w1kT_ref[...], ele,
                             preferred_element_type=jnp.float32)
                     * (1.0 / np.sqrt(nb))) * _SILU_C
    hv = jax.nn.silu(jnp.dot(w1vT_ref[...], ele,
                             preferred_element_type=jnp.float32)
                     * (1.0 / np.sqrt(nb))) * _SILU_C
    tdT = td_ref[...].T
    fsT = fs_ref[...].T
    fd = jnp.concatenate(
        [tdT[2 * mul * s:2 * mul * s + mul, :] for s in range(npack)], axis=1)
    qdd = jnp.concatenate(
        [tdT[2 * mul * s + mul:2 * mul * (s + 1), :] for s in range(npack)],
        axis=1)
    fs = jnp.concatenate(
        [fsT[2 * mul * s:2 * mul * s + mul, :] for s in range(npack)], axis=1)
    a_k = jnp.repeat(hk, mul, axis=0) * jnp.tile(fd, (mul, 1))
    a_v = jnp.repeat(hv, mul, axis=0) * jnp.tile(fs, (mul, 1))
    kv = jnp.dot(wc_ref[...], jnp.concatenate([a_k, a_v], axis=0),
                 preferred_element_type=jnp.float32)
    kv = kv * (shT_ref[...] * (1.0 / mul))
    k = kv[:mul, :]
    v = kv[mul:, :]
    logit = jnp.sum(qdd * k, axis=0, keepdims=True) * (1.0 / mul)
    x = 10.0 * (1.0 - lenT_ref[...] / _MAX_RADIUS)
    xs = jnp.where(x > 0.0, x, 1.0)
    cutoff = jnp.where(x > 0.0, jnp.exp(-1.0 / xs), 0.0)
    w = cutoff * jnp.exp(logit)
    eid = lax.broadcasted_iota(jnp.int32, (1, blk), 1) + pl.program_id(0) * blk
    w = jnp.where(eid < nedges, w, 0.0)
    u = jnp.sqrt(w) * v
    out32 = jnp.concatenate([u, jnp.broadcast_to(w, (mul, blk))], axis=0)
    stacked = jnp.concatenate(
        [out32[:, s * q:(s + 1) * q] for s in range(npack)], axis=0)
    o_ref[...] = stacked.T


def _edge_stage(eleT, td_pk, fs_pk, shT, lenT, W1k, W2k, W1v, W2v, nedges,
                blk=4096, interpret=False):
    nb, epad = eleT.shape
    mul = td_pk.shape[1] // 8
    hid = W1k.shape[1]
    w2kT = jnp.transpose(W2k.reshape(hid, mul, mul), (2, 0, 1)).reshape(mul, hid * mul)
    w2vT = jnp.transpose(W2v.reshape(hid, mul, mul), (2, 0, 1)).reshape(mul, hid * mul)
    zero = jnp.zeros((mul, hid * mul), jnp.float32)
    wc = jnp.concatenate([
        jnp.concatenate([w2kT, zero], axis=1),
        jnp.concatenate([zero, w2vT], axis=1),
    ], axis=0)
    grid = (epad // blk,)
    npack = 128 // (2 * mul)
    q = blk // npack
    return pl.pallas_call(
        functools.partial(_edge_body, nedges, mul),
        grid=grid,
        in_specs=[
            pl.BlockSpec((nb, blk), lambda i: (0, i)),
            pl.BlockSpec((q, 2 * mul * npack), lambda i: (i, 0)),
            pl.BlockSpec((q, 2 * mul * npack), lambda i: (i, 0)),
            pl.BlockSpec((1, blk), lambda i: (0, i)),
            pl.BlockSpec((1, blk), lambda i: (0, i)),
            pl.BlockSpec((mul, nb), lambda i: (0, 0)),
            pl.BlockSpec((mul, nb), lambda i: (0, 0)),
            pl.BlockSpec((2 * mul, 2 * hid * mul), lambda i: (0, 0)),
        ],
        out_specs=pl.BlockSpec((q, 2 * mul * npack), lambda i: (i, 0)),
        out_shape=jax.ShapeDtypeStruct((epad // npack, 2 * mul * npack),
                                       jnp.float32),
        interpret=interpret,
    )(eleT, td_pk, fs_pk, shT, lenT, W1k.T, W1v.T, wc)


def _norm_body(p0_ref, p1_ref, o_ref):
    mul = o_ref.shape[1]
    s = p0_ref[:, :mul] + p1_ref[:, :mul]
    z = p0_ref[:, mul:mul + 1] + p1_ref[:, mul:mul + 1]
    z = jnp.where(z == 0.0, 1.0, z)
    o_ref[...] = s * lax.rsqrt(z)


def _normalize(p0, p1, interpret=False):
    n = p0.shape[0]
    mul = p0.shape[1] // 2
    return pl.pallas_call(
        _norm_body,
        out_shape=jax.ShapeDtypeStruct((n, mul), jnp.float32),
        interpret=interpret,
    )(p0, p1)


def _gather_calls(table, dst3, src3, epad):
    n, tw = table.shape
    nw, j, ch = dst3.shape
    per = j * ch
    mesh = plsc.VectorSubcoreMesh(core_axis_name="c", subcore_axis_name="s")

    cpg = 5 if j % 5 == 0 else (4 if j % 4 == 0 else 1)
    ng = j // cpg
    rows_g = cpg * ch

    @functools.partial(
        pl.kernel, mesh=mesh,
        compiler_params=pltpu.CompilerParams(use_tc_tiling_on_sc=False),
        out_type=(jax.ShapeDtypeStruct((epad, tw), jnp.float32),
                  jax.ShapeDtypeStruct((epad, tw), jnp.float32)),
        scratch_types=[
            pltpu.VMEM((j, ch), jnp.int32),
            pltpu.VMEM((j, ch), jnp.int32),
            pltpu.VMEM((2 * rows_g, tw), jnp.float32),
            pltpu.VMEM((2 * rows_g, tw), jnp.float32),
            pltpu.SemaphoreType.DMA,
            pltpu.SemaphoreType.DMA,
        ],
    )
    def gk(t_hbm, d_hbm, s_hbm, td_out, fs_out,
           idx_d, idx_s, bt, bf, sg, sw):
        wid = lax.axis_index("s") * _NC + lax.axis_index("c")
        pltpu.sync_copy(d_hbm.at[wid], idx_d)
        pltpu.sync_copy(s_hbm.at[wid], idx_s)

        def issue_gets(g, b):
            gets = []
            for c in range(cpg):
                jj = g * cpg + c
                off = b * rows_g + c * ch
                gets.append(pltpu.async_copy(
                    t_hbm.at[idx_d.at[jj]], bt.at[pl.ds(off, ch)], sg))
                gets.append(pltpu.async_copy(
                    t_hbm.at[idx_s.at[jj]], bf.at[pl.ds(off, ch)], sg))
            return gets

        gets = {0: issue_gets(0, 0), 1: []}
        writes = {0: [], 1: []}
        for g in range(ng):
            b = g % 2
            o = (g + 1) % 2
            if g + 1 < ng:
                for h in writes[o]:
                    h.wait()
                gets[o] = issue_gets(g + 1, o)
            for h in gets[b]:
                h.wait()
            base = wid * per + g * rows_g
            writes[b] = [
                pltpu.async_copy(bt.at[pl.ds(b * rows_g, rows_g)],
                                 td_out.at[pl.ds(base, rows_g)], sw),
                pltpu.async_copy(bf.at[pl.ds(b * rows_g, rows_g)],
                                 fs_out.at[pl.ds(base, rows_g)], sw),
            ]
        for b in (0, 1):
            for h in writes[b]:
                h.wait()

    return gk(table, dst3, src3)


def _scatter_call(rows, dst3, zeros, n):
    epad, tw = rows.shape
    nw, j, ch = dst3.shape
    per = j * ch
    mesh = plsc.VectorSubcoreMesh(core_axis_name="c", subcore_axis_name="s")

    cpg = 8 if j % 8 == 0 else (10 if j % 10 == 0 else (5 if j % 5 == 0 else 1))
    ng = j // cpg
    rows_g = cpg * ch

    @functools.partial(
        pl.kernel, mesh=mesh,
        compiler_params=pltpu.CompilerParams(use_tc_tiling_on_sc=False),
        out_type=jax.ShapeDtypeStruct((_NC, n, tw), jnp.float32),
        scratch_types=[
            pltpu.VMEM((j, ch), jnp.int32),
            pltpu.VMEM((2 * rows_g, tw), jnp.float32),
            pltpu.VMEM_SHARED((n, tw), jnp.float32),
            pltpu.SemaphoreType.DMA,
            pltpu.SemaphoreType.DMA,
        ],
    )
    def sk(r_hbm, d_hbm, z_hbm, p_out, idx_v, buf, acc, sr, ss):
        cid = lax.axis_index("c")
        sid = lax.axis_index("s")
        wid = sid * _NC + cid
        pltpu.sync_copy(d_hbm.at[wid], idx_v)

        @pl.when(sid == 0)
        def _():
            pltpu.sync_copy(z_hbm, acc)

        plsc.subcore_barrier()

        def issue_read(g, b):
            base = wid * per + g * rows_g
            return [pltpu.async_copy(
                r_hbm.at[pl.ds(base, rows_g)],
                buf.at[pl.ds(b * rows_g, rows_g)], sr)]

        reads = {0: issue_read(0, 0), 1: []}
        scats = {0: [], 1: []}
        for g in range(ng):
            b = g % 2
            o = (g + 1) % 2
            if g + 1 < ng:
                for h in scats[o]:
                    h.wait()
                reads[o] = issue_read(g + 1, o)
            for h in reads[b]:
                h.wait()
            scats[b] = []
            for c in range(cpg):
                jj = g * cpg + c
                scats[b].append(pltpu.async_copy(
                    buf.at[pl.ds(b * rows_g + c * ch, ch)],
                    acc.at[idx_v.at[jj]], ss, add=True))
        for b in (0, 1):
            for h in scats[b]:
                h.wait()
        plsc.subcore_barrier()

        @pl.when(sid == 0)
        def _():
            pltpu.sync_copy(acc, p_out.at[cid])

    return sk(rows, dst3, zeros)


def kernel(f, edge_index, edge_length, edge_sh, edge_length_embedded,
           Wq, fck_W1, fck_W2, fcv_W1, fcv_W2, dot_W):
    n, mul = f.shape
    e = edge_index.shape[1]
    nb = edge_length_embedded.shape[1]
    grain = _NW * _CH
    epad = ((e + grain - 1) // grain) * grain
    pad = epad - e
    per = epad // _NW
    j = per // _CH

    blk = 4096
    npack = 128 // (2 * mul)
    q = blk // npack

    def _sc_order(ix):
        return jnp.transpose(ix.reshape(epad // blk, npack, q),
                             (0, 2, 1)).reshape(_NW, j, _CH)

    src = _sc_order(jnp.pad(edge_index[0], (0, pad)))
    dst = _sc_order(jnp.pad(edge_index[1], (0, pad)))

    eleT = jnp.pad(edge_length_embedded, ((0, pad), (0, 0))).T
    lenT = jnp.pad(edge_length, (0, pad)).reshape(1, epad)
    shT = jnp.pad(edge_sh[:, 0], (0, pad)).reshape(1, epad)

    table = _make_table(f, Wq, dot_W)
    td, fs = _gather_calls(table, dst, src, epad)
    rows_pk = _edge_stage(eleT, td.reshape(epad // npack, 128),
                          fs.reshape(epad // npack, 128), shT, lenT,
                          fck_W1, fck_W2, fcv_W1, fcv_W2, e, blk=blk)
    parts = _scatter_call(rows_pk.reshape(epad, 2 * mul), dst,
                          jnp.zeros((n, 2 * mul), jnp.float32), n)
    return _normalize(parts[0], parts[1])


# speedup vs baseline: 1.5522x; 1.1922x over previous
import functools

import jax
import jax.numpy as jnp
import numpy as np
from jax import lax
from jax.experimental import pallas as pl
from jax.experimental.pallas import tpu as pltpu
from jax.experimental.pallas import tpu_sc as plsc

_MAX_RADIUS = 1.3

_NC = 2
_NS = 16
_NW = _NC * _NS
_CH = 128


def _silu_c():
    z = np.linspace(-12.0, 12.0, 200001)
    phi = np.exp(-0.5 * z * z) / np.sqrt(2.0 * np.pi)
    s = z / (1.0 + np.exp(-z))
    return float(1.0 / np.sqrt(np.trapz(s * s * phi, z)))


_SILU_C = _silu_c()


def _table_body(f_ref, wq_ref, dw_ref, o_ref):
    f = f_ref[...]
    q = jnp.dot(f, wq_ref[...], preferred_element_type=jnp.float32)
    qd = jnp.dot(q, dw_ref[...], preferred_element_type=jnp.float32) * 0.25
    o_ref[...] = jnp.concatenate([f, qd], axis=1)


def _make_table(f, Wq, dot_W, interpret=False):
    n, mul = f.shape
    return pl.pallas_call(
        _table_body,
        out_shape=jax.ShapeDtypeStruct((n, 2 * mul), jnp.float32),
        interpret=interpret,
    )(f, Wq, dot_W)


def _edge_body(nedges, mul, eleT_ref, td_ref, fs_ref, shT_ref, lenT_ref,
               w1kT_ref, w1vT_ref, wc_ref, o_ref):
    blk = eleT_ref.shape[1]
    nb = eleT_ref.shape[0]
    npack = 128 // (2 * mul)
    q = blk // npack
    ele = eleT_ref[...]
    hk = jax.nn.silu(jnp.dot(w1kT_ref[...], ele,
                             preferred_element_type=jnp.float32)
                     * (1.0 / np.sqrt(nb))) * _SILU_C
    hv = jax.nn.silu(jnp.dot(w1vT_ref[...], ele,
                             preferred_element_type=jnp.float32)
                     * (1.0 / np.sqrt(nb))) * _SILU_C
    tdT = td_ref[...].T
    fsT = fs_ref[...].T
    fd = jnp.concatenate(
        [tdT[2 * mul * s:2 * mul * s + mul, :] for s in range(npack)], axis=1)
    qdd = jnp.concatenate(
        [tdT[2 * mul * s + mul:2 * mul * (s + 1), :] for s in range(npack)],
        axis=1)
    fs = jnp.concatenate(
        [fsT[mul * s:mul * (s + 1), :] for s in range(128 // mul)], axis=1)
    a_k = jnp.repeat(hk, mul, axis=0) * jnp.tile(fd, (mul, 1))
    a_v = jnp.repeat(hv, mul, axis=0) * jnp.tile(fs, (mul, 1))
    kv = jnp.dot(wc_ref[...], jnp.concatenate([a_k, a_v], axis=0),
                 preferred_element_type=jnp.float32)
    kv = kv * (shT_ref[...] * (1.0 / mul))
    k = kv[:mul, :]
    v = kv[mul:, :]
    logit = jnp.sum(qdd * k, axis=0, keepdims=True) * (1.0 / mul)
    x = 10.0 * (1.0 - lenT_ref[...] / _MAX_RADIUS)
    xs = jnp.where(x > 0.0, x, 1.0)
    cutoff = jnp.where(x > 0.0, jnp.exp(-1.0 / xs), 0.0)
    w = cutoff * jnp.exp(logit)
    eid = lax.broadcasted_iota(jnp.int32, (1, blk), 1) + pl.program_id(0) * blk
    w = jnp.where(eid < nedges, w, 0.0)
    u = jnp.sqrt(w) * v
    out32 = jnp.concatenate([u, jnp.broadcast_to(w, (mul, blk))], axis=0)
    stacked = jnp.concatenate(
        [out32[:, s * q:(s + 1) * q] for s in range(npack)], axis=0)
    o_ref[...] = stacked.T


def _edge_stage(eleT, td_pk, fs_pk, shT, lenT, W1k, W2k, W1v, W2v, nedges,
                blk=4096, interpret=False):
    nb, epad = eleT.shape
    mul = td_pk.shape[1] // 8
    hid = W1k.shape[1]
    w2kT = jnp.transpose(W2k.reshape(hid, mul, mul), (2, 0, 1)).reshape(mul, hid * mul)
    w2vT = jnp.transpose(W2v.reshape(hid, mul, mul), (2, 0, 1)).reshape(mul, hid * mul)
    zero = jnp.zeros((mul, hid * mul), jnp.float32)
    wc = jnp.concatenate([
        jnp.concatenate([w2kT, zero], axis=1),
        jnp.concatenate([zero, w2vT], axis=1),
    ], axis=0)
    grid = (epad // blk,)
    npack = 128 // (2 * mul)
    q = blk // npack
    return pl.pallas_call(
        functools.partial(_edge_body, nedges, mul),
        grid=grid,
        in_specs=[
            pl.BlockSpec((nb, blk), lambda i: (0, i)),
            pl.BlockSpec((q, 2 * mul * npack), lambda i: (i, 0)),
            pl.BlockSpec((blk * mul // 128, 128), lambda i: (i, 0)),
            pl.BlockSpec((1, blk), lambda i: (0, i)),
            pl.BlockSpec((1, blk), lambda i: (0, i)),
            pl.BlockSpec((mul, nb), lambda i: (0, 0)),
            pl.BlockSpec((mul, nb), lambda i: (0, 0)),
            pl.BlockSpec((2 * mul, 2 * hid * mul), lambda i: (0, 0)),
        ],
        out_specs=pl.BlockSpec((q, 2 * mul * npack), lambda i: (i, 0)),
        out_shape=jax.ShapeDtypeStruct((epad // npack, 2 * mul * npack),
                                       jnp.float32),
        interpret=interpret,
    )(eleT, td_pk, fs_pk, shT, lenT, W1k.T, W1v.T, wc)


def _norm_body(p0_ref, p1_ref, o_ref):
    mul = o_ref.shape[1]
    s = p0_ref[:, :mul] + p1_ref[:, :mul]
    z = p0_ref[:, mul:mul + 1] + p1_ref[:, mul:mul + 1]
    z = jnp.where(z == 0.0, 1.0, z)
    o_ref[...] = s * lax.rsqrt(z)


def _normalize(p0, p1, interpret=False):
    n = p0.shape[0]
    mul = p0.shape[1] // 2
    return pl.pallas_call(
        _norm_body,
        out_shape=jax.ShapeDtypeStruct((n, mul), jnp.float32),
        interpret=interpret,
    )(p0, p1)


def _gather_calls(table, f, dst3, src3, epad):
    n, tw = table.shape
    fw = f.shape[1]
    nw, j, ch = dst3.shape
    per = j * ch
    mesh = plsc.VectorSubcoreMesh(core_axis_name="c", subcore_axis_name="s")

    cpg = 8 if j % 8 == 0 else (5 if j % 5 == 0 else 1)
    ng = j // cpg
    rows_g = cpg * ch

    @functools.partial(
        pl.kernel, mesh=mesh,
        compiler_params=pltpu.CompilerParams(use_tc_tiling_on_sc=False),
        out_type=(jax.ShapeDtypeStruct((epad, tw), jnp.float32),
                  jax.ShapeDtypeStruct((epad, fw), jnp.float32)),
        scratch_types=[
            pltpu.VMEM((j, ch), jnp.int32),
            pltpu.VMEM((j, ch), jnp.int32),
            pltpu.VMEM((2 * rows_g, tw), jnp.float32),
            pltpu.VMEM((2 * rows_g, fw), jnp.float32),
            pltpu.SemaphoreType.DMA,
            pltpu.SemaphoreType.DMA,
        ],
    )
    def gk(t_hbm, f_hbm, d_hbm, s_hbm, td_out, fs_out,
           idx_d, idx_s, bt, bf, sg, sw):
        wid = lax.axis_index("s") * _NC + lax.axis_index("c")
        pltpu.sync_copy(d_hbm.at[wid], idx_d)
        pltpu.sync_copy(s_hbm.at[wid], idx_s)

        def issue_gets(g, b):
            gets = []
            for c in range(cpg):
                jj = g * cpg + c
                off = b * rows_g + c * ch
                gets.append(pltpu.async_copy(
                    t_hbm.at[idx_d.at[jj]], bt.at[pl.ds(off, ch)], sg))
                gets.append(pltpu.async_copy(
                    f_hbm.at[idx_s.at[jj]], bf.at[pl.ds(off, ch)], sg))
            return gets

        gets = {0: issue_gets(0, 0), 1: []}
        writes = {0: [], 1: []}
        for g in range(ng):
            b = g % 2
            o = (g + 1) % 2
            if g + 1 < ng:
                for h in writes[o]:
                    h.wait()
                gets[o] = issue_gets(g + 1, o)
            for h in gets[b]:
                h.wait()
            base = wid * per + g * rows_g
            writes[b] = [
                pltpu.async_copy(bt.at[pl.ds(b * rows_g, rows_g)],
                                 td_out.at[pl.ds(base, rows_g)], sw),
                pltpu.async_copy(bf.at[pl.ds(b * rows_g, rows_g)],
                                 fs_out.at[pl.ds(base, rows_g)], sw),
            ]
        for b in (0, 1):
            for h in writes[b]:
                h.wait()

    return gk(table, f, dst3, src3)


def _scatter_call(rows, dst3, zeros, n):
    epad, tw = rows.shape
    nw, j, ch = dst3.shape
    per = j * ch
    mesh = plsc.VectorSubcoreMesh(core_axis_name="c", subcore_axis_name="s")

    cpg = 8 if j % 8 == 0 else (10 if j % 10 == 0 else (5 if j % 5 == 0 else 1))
    ng = j // cpg
    rows_g = cpg * ch

    @functools.partial(
        pl.kernel, mesh=mesh,
        compiler_params=pltpu.CompilerParams(use_tc_tiling_on_sc=False),
        out_type=jax.ShapeDtypeStruct((_NC, n, tw), jnp.float32),
        scratch_types=[
            pltpu.VMEM((j, ch), jnp.int32),
            pltpu.VMEM((2 * rows_g, tw), jnp.float32),
            pltpu.VMEM_SHARED((n, tw), jnp.float32),
            pltpu.SemaphoreType.DMA,
            pltpu.SemaphoreType.DMA,
        ],
    )
    def sk(r_hbm, d_hbm, z_hbm, p_out, idx_v, buf, acc, sr, ss):
        cid = lax.axis_index("c")
        sid = lax.axis_index("s")
        wid = sid * _NC + cid
        pltpu.sync_copy(d_hbm.at[wid], idx_v)

        @pl.when(sid == 0)
        def _():
            pltpu.sync_copy(z_hbm, acc)

        plsc.subcore_barrier()

        def issue_read(g, b):
            base = wid * per + g * rows_g
            return [pltpu.async_copy(
                r_hbm.at[pl.ds(base, rows_g)],
                buf.at[pl.ds(b * rows_g, rows_g)], sr)]

        reads = {0: issue_read(0, 0), 1: []}
        scats = {0: [], 1: []}
        for g in range(ng):
            b = g % 2
            o = (g + 1) % 2
            if g + 1 < ng:
                for h in scats[o]:
                    h.wait()
                reads[o] = issue_read(g + 1, o)
            for h in reads[b]:
                h.wait()
            scats[b] = []
            for c in range(cpg):
                jj = g * cpg + c
                scats[b].append(pltpu.async_copy(
                    buf.at[pl.ds(b * rows_g + c * ch, ch)],
                    acc.at[idx_v.at[jj]], ss, add=True))
        for b in (0, 1):
            for h in scats[b]:
                h.wait()
        plsc.subcore_barrier()

        @pl.when(sid == 0)
        def _():
            pltpu.sync_copy(acc, p_out.at[cid])

    return sk(rows, dst3, zeros)


def kernel(f, edge_index, edge_length, edge_sh, edge_length_embedded,
           Wq, fck_W1, fck_W2, fcv_W1, fcv_W2, dot_W):
    n, mul = f.shape
    e = edge_index.shape[1]
    nb = edge_length_embedded.shape[1]
    grain = _NW * _CH
    epad = ((e + grain - 1) // grain) * grain
    pad = epad - e
    per = epad // _NW
    j = per // _CH

    blk = 4096
    npack = 128 // (2 * mul)
    q = blk // npack

    def _sc_order(ix, np_):
        return jnp.transpose(ix.reshape(epad // blk, np_, blk // np_),
                             (0, 2, 1)).reshape(_NW, j, _CH)

    src = _sc_order(jnp.pad(edge_index[0], (0, pad)), 128 // mul)
    dst = _sc_order(jnp.pad(edge_index[1], (0, pad)), npack)

    eleT = jnp.pad(edge_length_embedded, ((0, pad), (0, 0))).T
    lenT = jnp.pad(edge_length, (0, pad)).reshape(1, epad)
    shT = jnp.pad(edge_sh[:, 0], (0, pad)).reshape(1, epad)

    table = _make_table(f, Wq, dot_W)
    td, fs = _gather_calls(table, f, dst, src, epad)
    rows_pk = _edge_stage(eleT, td.reshape(epad // npack, 128),
                          fs.reshape(epad * mul // 128, 128), shT, lenT,
                          fck_W1, fck_W2, fcv_W1, fcv_W2, e, blk=blk)
    parts = _scatter_call(rows_pk.reshape(epad, 2 * mul), dst,
                          jnp.zeros((n, 2 * mul), jnp.float32), n)
    return _normalize(parts[0], parts[1])


# speedup vs baseline: 1.5651x; 1.0083x over previous
import functools

import jax
import jax.numpy as jnp
import numpy as np
from jax import lax
from jax.experimental import pallas as pl
from jax.experimental.pallas import tpu as pltpu
from jax.experimental.pallas import tpu_sc as plsc

_MAX_RADIUS = 1.3

_NC = 2
_NS = 16
_NW = _NC * _NS
_CH = 128


def _silu_c():
    z = np.linspace(-12.0, 12.0, 200001)
    phi = np.exp(-0.5 * z * z) / np.sqrt(2.0 * np.pi)
    s = z / (1.0 + np.exp(-z))
    return float(1.0 / np.sqrt(np.trapz(s * s * phi, z)))


_SILU_C = _silu_c()


def _table_body(f_ref, wq_ref, dw_ref, o_ref):
    f = f_ref[...]
    q = jnp.dot(f, wq_ref[...], preferred_element_type=jnp.float32)
    qd = jnp.dot(q, dw_ref[...], preferred_element_type=jnp.float32) * 0.25
    o_ref[...] = jnp.concatenate([f, qd], axis=1)


def _make_table(f, Wq, dot_W, interpret=False):
    n, mul = f.shape
    return pl.pallas_call(
        _table_body,
        out_shape=jax.ShapeDtypeStruct((n, 2 * mul), jnp.float32),
        interpret=interpret,
    )(f, Wq, dot_W)


def _hid_body(nb, h_scale, eleT_ref, w1kT_ref, w1vT_ref, o_ref):
    ele = eleT_ref[...]
    hk = jax.nn.silu(jnp.dot(w1kT_ref[...], ele,
                             preferred_element_type=jnp.float32)
                     * (1.0 / np.sqrt(nb))) * h_scale
    hv = jax.nn.silu(jnp.dot(w1vT_ref[...], ele,
                             preferred_element_type=jnp.float32)
                     * (1.0 / np.sqrt(nb))) * h_scale
    o_ref[...] = jnp.concatenate([hk, hv], axis=0)


def _hid_stage(eleT, W1k, W1v, blk=4096, interpret=False):
    nb, epad = eleT.shape
    hid = W1k.shape[1]
    return pl.pallas_call(
        functools.partial(_hid_body, nb, _SILU_C),
        grid=(epad // blk,),
        in_specs=[
            pl.BlockSpec((nb, blk), lambda i: (0, i)),
            pl.BlockSpec((hid, nb), lambda i: (0, 0)),
            pl.BlockSpec((hid, nb), lambda i: (0, 0)),
        ],
        out_specs=pl.BlockSpec((2 * hid, blk), lambda i: (0, i)),
        out_shape=jax.ShapeDtypeStruct((2 * hid, epad), jnp.float32),
        interpret=interpret,
    )(eleT, W1k.T, W1v.T)


def _edge_body(nedges, mul, h_ref, td_ref, fs_ref, shT_ref, lenT_ref,
               wc_ref, o_ref):
    blk = h_ref.shape[1]
    hid2 = h_ref.shape[0]
    npack = 128 // (2 * mul)
    q = blk // npack
    hk = h_ref[:hid2 // 2, :]
    hv = h_ref[hid2 // 2:, :]
    tdT = td_ref[...].T
    fsT = fs_ref[...].T
    fd = jnp.concatenate(
        [tdT[2 * mul * s:2 * mul * s + mul, :] for s in range(npack)], axis=1)
    qdd = jnp.concatenate(
        [tdT[2 * mul * s + mul:2 * mul * (s + 1), :] for s in range(npack)],
        axis=1)
    fs = jnp.concatenate(
        [fsT[mul * s:mul * (s + 1), :] for s in range(128 // mul)], axis=1)
    a_k = jnp.repeat(hk, mul, axis=0) * jnp.tile(fd, (mul, 1))
    a_v = jnp.repeat(hv, mul, axis=0) * jnp.tile(fs, (mul, 1))
    kv = jnp.dot(wc_ref[...], jnp.concatenate([a_k, a_v], axis=0),
                 preferred_element_type=jnp.float32)
    kv = kv * (shT_ref[...] * (1.0 / mul))
    k = kv[:mul, :]
    v = kv[mul:, :]
    logit = jnp.sum(qdd * k, axis=0, keepdims=True) * (1.0 / mul)
    x = 10.0 * (1.0 - lenT_ref[...] / _MAX_RADIUS)
    xs = jnp.where(x > 0.0, x, 1.0)
    cutoff = jnp.where(x > 0.0, jnp.exp(-1.0 / xs), 0.0)
    w = cutoff * jnp.exp(logit)
    eid = lax.broadcasted_iota(jnp.int32, (1, blk), 1) + pl.program_id(0) * blk
    w = jnp.where(eid < nedges, w, 0.0)
    u = jnp.sqrt(w) * v
    out32 = jnp.concatenate([u, jnp.broadcast_to(w, (mul, blk))], axis=0)
    stacked = jnp.concatenate(
        [out32[:, s * q:(s + 1) * q] for s in range(npack)], axis=0)
    o_ref[...] = stacked.T


def _edge_stage(hkv, td_pk, fs_pk, shT, lenT, W2k, W2v, nedges,
                blk=4096, interpret=False):
    hid2, epad = hkv.shape
    mul = td_pk.shape[1] // 8
    hid = hid2 // 2
    w2kT = jnp.transpose(W2k.reshape(hid, mul, mul), (2, 0, 1)).reshape(mul, hid * mul)
    w2vT = jnp.transpose(W2v.reshape(hid, mul, mul), (2, 0, 1)).reshape(mul, hid * mul)
    zero = jnp.zeros((mul, hid * mul), jnp.float32)
    wc = jnp.concatenate([
        jnp.concatenate([w2kT, zero], axis=1),
        jnp.concatenate([zero, w2vT], axis=1),
    ], axis=0)
    grid = (epad // blk,)
    npack = 128 // (2 * mul)
    q = blk // npack
    return pl.pallas_call(
        functools.partial(_edge_body, nedges, mul),
        grid=grid,
        in_specs=[
            pl.BlockSpec((hid2, blk), lambda i: (0, i)),
            pl.BlockSpec((q, 2 * mul * npack), lambda i: (i, 0)),
            pl.BlockSpec((blk * mul // 128, 128), lambda i: (i, 0)),
            pl.BlockSpec((1, blk), lambda i: (0, i)),
            pl.BlockSpec((1, blk), lambda i: (0, i)),
            pl.BlockSpec((2 * mul, 2 * hid * mul), lambda i: (0, 0)),
        ],
        out_specs=pl.BlockSpec((q, 2 * mul * npack), lambda i: (i, 0)),
        out_shape=jax.ShapeDtypeStruct((epad // npack, 2 * mul * npack),
                                       jnp.float32),
        interpret=interpret,
    )(hkv, td_pk, fs_pk, shT, lenT, wc)


def _norm_body(p0_ref, p1_ref, o_ref):
    mul = o_ref.shape[1]
    s = p0_ref[:, :mul] + p1_ref[:, :mul]
    z = p0_ref[:, mul:mul + 1] + p1_ref[:, mul:mul + 1]
    z = jnp.where(z == 0.0, 1.0, z)
    o_ref[...] = s * lax.rsqrt(z)


def _normalize(p0, p1, interpret=False):
    n = p0.shape[0]
    mul = p0.shape[1] // 2
    return pl.pallas_call(
        _norm_body,
        out_shape=jax.ShapeDtypeStruct((n, mul), jnp.float32),
        interpret=interpret,
    )(p0, p1)


def _gather_calls(table, f, dst3, src3, epad):
    n, tw = table.shape
    fw = f.shape[1]
    nw, j, ch = dst3.shape
    per = j * ch
    mesh = plsc.VectorSubcoreMesh(core_axis_name="c", subcore_axis_name="s")

    cpg = 8 if j % 8 == 0 else (5 if j % 5 == 0 else 1)
    ng = j // cpg
    rows_g = cpg * ch

    @functools.partial(
        pl.kernel, mesh=mesh,
        compiler_params=pltpu.CompilerParams(use_tc_tiling_on_sc=False),
        out_type=(jax.ShapeDtypeStruct((epad, tw), jnp.float32),
                  jax.ShapeDtypeStruct((epad, fw), jnp.float32)),
        scratch_types=[
            pltpu.VMEM((j, ch), jnp.int32),
            pltpu.VMEM((j, ch), jnp.int32),
            pltpu.VMEM((2 * rows_g, tw), jnp.float32),
            pltpu.VMEM((2 * rows_g, fw), jnp.float32),
            pltpu.SemaphoreType.DMA,
            pltpu.SemaphoreType.DMA,
        ],
    )
    def gk(t_hbm, f_hbm, d_hbm, s_hbm, td_out, fs_out,
           idx_d, idx_s, bt, bf, sg, sw):
        wid = lax.axis_index("s") * _NC + lax.axis_index("c")
        pltpu.sync_copy(d_hbm.at[wid], idx_d)
        pltpu.sync_copy(s_hbm.at[wid], idx_s)

        def issue_gets(g, b):
            gets = []
            for c in range(cpg):
                jj = g * cpg + c
                off = b * rows_g + c * ch
                gets.append(pltpu.async_copy(
                    t_hbm.at[idx_d.at[jj]], bt.at[pl.ds(off, ch)], sg))
                gets.append(pltpu.async_copy(
                    f_hbm.at[idx_s.at[jj]], bf.at[pl.ds(off, ch)], sg))
            return gets

        gets = {0: issue_gets(0, 0), 1: []}
        writes = {0: [], 1: []}
        for g in range(ng):
            b = g % 2
            o = (g + 1) % 2
            if g + 1 < ng:
                for h in writes[o]:
                    h.wait()
                gets[o] = issue_gets(g + 1, o)
            for h in gets[b]:
                h.wait()
            base = wid * per + g * rows_g
            writes[b] = [
                pltpu.async_copy(bt.at[pl.ds(b * rows_g, rows_g)],
                                 td_out.at[pl.ds(base, rows_g)], sw),
                pltpu.async_copy(bf.at[pl.ds(b * rows_g, rows_g)],
                                 fs_out.at[pl.ds(base, rows_g)], sw),
            ]
        for b in (0, 1):
            for h in writes[b]:
                h.wait()

    return gk(table, f, dst3, src3)


def _scatter_call(rows, dst3, zeros, n):
    epad, tw = rows.shape
    nw, j, ch = dst3.shape
    per = j * ch
    mesh = plsc.VectorSubcoreMesh(core_axis_name="c", subcore_axis_name="s")

    cpg = 8 if j % 8 == 0 else (10 if j % 10 == 0 else (5 if j % 5 == 0 else 1))
    ng = j // cpg
    rows_g = cpg * ch

    @functools.partial(
        pl.kernel, mesh=mesh,
        compiler_params=pltpu.CompilerParams(use_tc_tiling_on_sc=False),
        out_type=jax.ShapeDtypeStruct((_NC, n, tw), jnp.float32),
        scratch_types=[
            pltpu.VMEM((j, ch), jnp.int32),
            pltpu.VMEM((2 * rows_g, tw), jnp.float32),
            pltpu.VMEM_SHARED((n, tw), jnp.float32),
            pltpu.SemaphoreType.DMA,
            pltpu.SemaphoreType.DMA,
        ],
    )
    def sk(r_hbm, d_hbm, z_hbm, p_out, idx_v, buf, acc, sr, ss):
        cid = lax.axis_index("c")
        sid = lax.axis_index("s")
        wid = sid * _NC + cid
        pltpu.sync_copy(d_hbm.at[wid], idx_v)

        @pl.when(sid == 0)
        def _():
            pltpu.sync_copy(z_hbm, acc)

        plsc.subcore_barrier()

        def issue_read(g, b):
            base = wid * per + g * rows_g
            return [pltpu.async_copy(
                r_hbm.at[pl.ds(base, rows_g)],
                buf.at[pl.ds(b * rows_g, rows_g)], sr)]

        reads = {0: issue_read(0, 0), 1: []}
        scats = {0: [], 1: []}
        for g in range(ng):
            b = g % 2
            o = (g + 1) % 2
            if g + 1 < ng:
                for h in scats[o]:
                    h.wait()
                reads[o] = issue_read(g + 1, o)
            for h in reads[b]:
                h.wait()
            scats[b] = []
            for c in range(cpg):
                jj = g * cpg + c
                scats[b].append(pltpu.async_copy(
                    buf.at[pl.ds(b * rows_g + c * ch, ch)],
                    acc.at[idx_v.at[jj]], ss, add=True))
        for b in (0, 1):
            for h in scats[b]:
                h.wait()
        plsc.subcore_barrier()

        @pl.when(sid == 0)
        def _():
            pltpu.sync_copy(acc, p_out.at[cid])

    return sk(rows, dst3, zeros)


def kernel(f, edge_index, edge_length, edge_sh, edge_length_embedded,
           Wq, fck_W1, fck_W2, fcv_W1, fcv_W2, dot_W):
    n, mul = f.shape
    e = edge_index.shape[1]
    nb = edge_length_embedded.shape[1]
    grain = _NW * _CH
    epad = ((e + grain - 1) // grain) * grain
    pad = epad - e
    per = epad // _NW
    j = per // _CH

    blk = 4096
    npack = 128 // (2 * mul)
    q = blk // npack

    def _sc_order(ix, np_):
        return jnp.transpose(ix.reshape(epad // blk, np_, blk // np_),
                             (0, 2, 1)).reshape(_NW, j, _CH)

    src = _sc_order(jnp.pad(edge_index[0], (0, pad)), 128 // mul)
    dst = _sc_order(jnp.pad(edge_index[1], (0, pad)), npack)

    eleT = jnp.pad(edge_length_embedded, ((0, pad), (0, 0))).T
    lenT = jnp.pad(edge_length, (0, pad)).reshape(1, epad)
    shT = jnp.pad(edge_sh[:, 0], (0, pad)).reshape(1, epad)

    table = _make_table(f, Wq, dot_W)
    td, fs = _gather_calls(table, f, dst, src, epad)
    hkv = _hid_stage(eleT, fck_W1, fcv_W1, blk=blk)
    rows_pk = _edge_stage(hkv, td.reshape(epad // npack, 128),
                          fs.reshape(epad * mul // 128, 128), shT, lenT,
                          fck_W2, fcv_W2, e, blk=blk)
    parts = _scatter_call(rows_pk.reshape(epad, 2 * mul), dst,
                          jnp.zeros((n, 2 * mul), jnp.float32), n)
    return _normalize(parts[0], parts[1])
